# Initial kernel scaffold; baseline (speedup 1.0000x reference)
import jax
import jax.numpy as jnp
from jax.experimental import pallas as pl


def kernel(x, edge_index, W1, b1, ln1_g, ln1_b, W2, b2, ln2_g, ln2_b, Wout, bout):
    raise NotImplementedError("write your pallas kernel here")


# speedup vs baseline: 5.0227x; 5.0227x over previous
import functools

import jax
import jax.numpy as jnp
from jax import lax
from jax.experimental import pallas as pl
from jax.experimental.pallas import tpu as pltpu
from jax.experimental.pallas import tpu_sc as plsc

N = 10000
E = 320000
D = 128
WX = 144
NP = 10240
T = 10.0
ENT_W = 1.0
EPS = 1e-12

NC = 2
NS = 16
NWORK = NC * NS
CHUNK = 128
CH_PER = -(-E // (CHUNK * NWORK) // 8) * 8
CH_TOT = CH_PER * NWORK
EPAD = CH_TOT * CHUNK
GARBAGE = 10200
SLICE = NP // NS

BR = 256
NBLK = NP // BR


def _sc_segsum(W):
    mesh = plsc.VectorSubcoreMesh(core_axis_name="c", subcore_axis_name="s")

    @functools.partial(
        pl.kernel,
        out_type=jax.ShapeDtypeStruct((2 * NP, W), jnp.float32),
        mesh=mesh,
        compiler_params=pltpu.CompilerParams(use_tc_tiling_on_sc=False),
        scratch_types=[
            pltpu.VMEM((CH_PER, CHUNK), jnp.int32),
            pltpu.VMEM((CH_PER, CHUNK), jnp.int32),
            pltpu.VMEM((CHUNK, W), jnp.float32),
            pltpu.VMEM_SHARED((NP, W), jnp.float32),
            pltpu.SemaphoreType.DMA,
        ],
    )
    def k(table, gidx, sidx, zrows, out, gidx_v, sidx_v, rows_v, acc, sem):
        c = lax.axis_index("c")
        s = lax.axis_index("s")
        tile = c * NS + s
        base = tile * CH_PER
        pltpu.sync_copy(gidx.at[pl.ds(base, CH_PER)], gidx_v)
        pltpu.sync_copy(sidx.at[pl.ds(base, CH_PER)], sidx_v)
        pltpu.sync_copy(zrows, acc.at[pl.ds(s * SLICE, SLICE)])
        plsc.subcore_barrier()

        def step(j, carry):
            pltpu.async_copy(table.at[gidx_v.at[j]], rows_v, sem).wait()
            pltpu.sync_copy(rows_v, acc.at[sidx_v.at[j]], add=True)
            return carry

        lax.fori_loop(0, CH_PER, step, 0)
        plsc.subcore_barrier()
        pltpu.sync_copy(acc.at[pl.ds(s * SLICE, SLICE)],
                        out.at[pl.ds(c * NP + s * SLICE, SLICE)])

    return k


def _sc_degree():
    mesh = plsc.VectorSubcoreMesh(core_axis_name="c", subcore_axis_name="s")

    @functools.partial(
        pl.kernel,
        out_type=jax.ShapeDtypeStruct((2 * NP, D), jnp.float32),
        mesh=mesh,
        compiler_params=pltpu.CompilerParams(use_tc_tiling_on_sc=False),
        scratch_types=[
            pltpu.VMEM((CH_PER, CHUNK), jnp.int32),
            pltpu.VMEM((CHUNK, D), jnp.float32),
            pltpu.VMEM_SHARED((NP, D), jnp.float32),
        ],
    )
    def k(ones_hbm, sidx, zrows, out, sidx_v, rows_v, acc):
        c = lax.axis_index("c")
        s = lax.axis_index("s")
        tile = c * NS + s
        base = tile * CH_PER
        pltpu.sync_copy(sidx.at[pl.ds(base, CH_PER)], sidx_v)
        pltpu.sync_copy(ones_hbm, rows_v)
        pltpu.sync_copy(zrows, acc.at[pl.ds(s * SLICE, SLICE)])
        plsc.subcore_barrier()

        def step(j, carry):
            pltpu.sync_copy(rows_v, acc.at[sidx_v.at[j]], add=True)
            return carry

        lax.fori_loop(0, CH_PER, step, 0)
        plsc.subcore_barrier()
        pltpu.sync_copy(acc.at[pl.ds(s * SLICE, SLICE)],
                        out.at[pl.ds(c * NP + s * SLICE, SLICE)])

    return k


def _tc_prep1(x_ref, w_ref, degp_ref, hp_ref, dinv_ref, deg0_ref):
    deg0 = degp_ref[0, :, 0:1] + degp_ref[1, :, 0:1]
    dinv = lax.rsqrt(deg0 + 1.0)
    deg0_ref[...] = deg0
    dinv_ref[...] = dinv
    h = jnp.dot(x_ref[...], w_ref[...], preferred_element_type=jnp.float32)
    hp_ref[...] = h * dinv


def _tc_conv_post(hp_ref, sp_ref, dinv_ref, b_ref, x_out_ref):
    s = sp_ref[0] + sp_ref[1]
    out = dinv_ref[...] * (s + hp_ref[...]) + b_ref[...]
    n = jnp.sum(out * out, axis=1, keepdims=True)
    x_out_ref[...] = jnp.concatenate(
        [out, n, jnp.zeros((out.shape[0], WX - D - 1), jnp.float32)], axis=1)


def _tc_energy(x_ref, fp_ref, deg0_ref, e_ref):
    xa = x_ref[...]
    x = xa[:, 0:D]
    n = xa[:, D:D + 1]
    f = fp_ref[0] + fp_ref[1]
    s = f[:, 0:D]
    t = f[:, D:D + 1]
    e_ref[...] = (0.5 * t + 0.5 * deg0_ref[...] * n
                  - jnp.sum(s * x, axis=1, keepdims=True))


def _tc_softgrad(e_ref, g_ref):
    lin = (lax.broadcasted_iota(jnp.int32, (NP // D, D), 0) * D
           + lax.broadcasted_iota(jnp.int32, (NP // D, D), 1))
    mask = lin < N
    e = jnp.where(mask, e_ref[...], 0.0)
    m = jnp.max(e)
    r = 1.0 / (m + EPS)
    z = -(e * r) / T
    zmax = jnp.max(jnp.where(mask, z, -1e30))
    ez = jnp.where(mask, jnp.exp(z - zmax), 0.0)
    p = ez / jnp.sum(ez)
    q = -(jnp.log(p + EPS) + p / (p + EPS))
    pq = jnp.sum(p * q)
    w = -(p * (q - pq)) / T
    onehot = jnp.where(mask & (e == m), 1.0, 0.0)
    cnt = jnp.sum(onehot)
    g_ref[...] = w * r - onehot * (jnp.sum(w * e) * r * r / cnt)


def _tc_ybuild(x_ref, g_ref, y_ref):
    xa = x_ref[...]
    x = xa[:, 0:D]
    ge = g_ref[...]
    y_ref[...] = jnp.concatenate(
        [ge * x, ge, jnp.zeros((x.shape[0], WX - D - 1), jnp.float32)],
        axis=1)


def _tc_post(x_ref, fp_ref, bp_ref, g_ref, deg0_ref, dinv_ref, lng_ref,
             lnb_ref, w_ref, hp_ref):
    xa = x_ref[...]
    x = xa[:, 0:D]
    f = fp_ref[0] + fp_ref[1]
    s = f[:, 0:D]
    bm = bp_ref[0] + bp_ref[1]
    cc = bm[:, 0:D]
    a = bm[:, D:D + 1]
    ge = g_ref[...]
    grad = x * (a + ge * deg0_ref[...]) - cc - ge * s
    h = x + ENT_W * grad
    h = jnp.maximum(h, 0.0)
    mu = jnp.mean(h, axis=1, keepdims=True)
    d = h - mu
    var = jnp.mean(d * d, axis=1, keepdims=True)
    hn = d / jnp.sqrt(var + 1e-5) * lng_ref[...] + lnb_ref[...]
    h2 = jnp.dot(hn, w_ref[...], preferred_element_type=jnp.float32)
    hp_ref[...] = h2 * dinv_ref[...]


def _tc_final(hp_ref, sp_ref, dinv_ref, b_ref, out_ref):
    s = sp_ref[0] + sp_ref[1]
    out_ref[...] = dinv_ref[...] * (s + hp_ref[...]) + b_ref[...]


def _sds(shape):
    return jax.ShapeDtypeStruct(shape, jnp.float32)


def _row(w):
    return pl.BlockSpec((BR, w), lambda i: (i, 0))


def _rowp(w):
    return pl.BlockSpec((2, BR, w), lambda i: (0, i, 0))


def _bcast(w):
    return pl.BlockSpec((1, w), lambda i: (0, 0))


def _mat():
    return pl.BlockSpec((D, D), lambda i: (0, 0))


def kernel(x, edge_index, W1, b1, ln1_g, ln1_b, W2, b2, ln2_g, ln2_b,
           Wout, bout):
    f32 = jnp.float32
    src = edge_index[0].astype(jnp.int32)
    dst = edge_index[1].astype(jnp.int32)
    pe = EPAD - E
    zpad = jnp.zeros((pe,), jnp.int32)
    gpad = jnp.full((pe,), GARBAGE, jnp.int32)
    g_src = jnp.concatenate([src, zpad]).reshape(CH_TOT, CHUNK)
    s_dst = jnp.concatenate([dst, gpad]).reshape(CH_TOT, CHUNK)
    g_dst = jnp.concatenate([dst, zpad]).reshape(CH_TOT, CHUNK)
    s_src = jnp.concatenate([src, gpad]).reshape(CH_TOT, CHUNK)
    zr128 = jnp.zeros((SLICE, D), f32)
    zr144 = jnp.zeros((SLICE, WX), f32)
    ones128 = jnp.ones((CHUNK, D), f32)
    xp = jnp.pad(x, ((0, NP - N), (0, 0)))
    b1r = b1.reshape(1, D)
    b2r = b2.reshape(1, D)
    boutr = bout.reshape(1, D)
    ln1gr = ln1_g.reshape(1, D)
    ln1br = ln1_b.reshape(1, D)
    ln2gr = ln2_g.reshape(1, D)
    ln2br = ln2_b.reshape(1, D)

    seg128 = _sc_segsum(D)
    seg144 = _sc_segsum(WX)

    degp = _sc_degree()(ones128, s_dst, zr128).reshape(2, NP, D)

    hp1, dinv, deg0 = pl.pallas_call(
        _tc_prep1,
        grid=(NBLK,),
        in_specs=[_row(D), _mat(), _rowp(D)],
        out_specs=[_row(D), _row(1), _row(1)],
        out_shape=[_sds((NP, D)), _sds((NP, 1)), _sds((NP, 1))],
    )(xp, W1, degp)

    def conv_post(hp, sp, br):
        return pl.pallas_call(
            _tc_conv_post,
            grid=(NBLK,),
            in_specs=[_row(D), _rowp(D), _row(1), _bcast(D)],
            out_specs=_row(WX),
            out_shape=_sds((NP, WX)),
        )(hp, sp, dinv, br)

    def entropy_grad(xa, fp):
        e = pl.pallas_call(
            _tc_energy,
            grid=(NBLK,),
            in_specs=[_row(WX), _rowp(WX), _row(1)],
            out_specs=_row(1),
            out_shape=_sds((NP, 1)),
        )(xa, fp, deg0)
        ge = pl.pallas_call(
            _tc_softgrad,
            out_shape=_sds((NP // D, D)),
        )(e.reshape(NP // D, D)).reshape(NP, 1)
        y = pl.pallas_call(
            _tc_ybuild,
            grid=(NBLK,),
            in_specs=[_row(WX), _row(1)],
            out_specs=_row(WX),
            out_shape=_sds((NP, WX)),
        )(xa, ge)
        return y, ge

    def post(xa, fp, bp, ge, lng, lnb, w):
        return pl.pallas_call(
            _tc_post,
            grid=(NBLK,),
            in_specs=[_row(WX), _rowp(WX), _rowp(WX), _row(1), _row(1),
                      _row(1), _bcast(D), _bcast(D), _mat()],
            out_specs=_row(D),
            out_shape=_sds((NP, D)),
        )(xa, fp, bp, ge, deg0, dinv, lng, lnb, w)

    s1 = seg128(hp1, g_src, s_dst, zr128).reshape(2, NP, D)
    X1 = conv_post(hp1, s1, b1r)
    F1 = seg144(X1, g_src, s_dst, zr144).reshape(2, NP, WX)
    Y1, g1 = entropy_grad(X1, F1)
    B1 = seg144(Y1, g_dst, s_src, zr144).reshape(2, NP, WX)
    hp2 = post(X1, F1, B1, g1, ln1gr, ln1br, W2)

    s2 = seg128(hp2, g_src, s_dst, zr128).reshape(2, NP, D)
    X2 = conv_post(hp2, s2, b2r)
    F2 = seg144(X2, g_src, s_dst, zr144).reshape(2, NP, WX)
    Y2, g2 = entropy_grad(X2, F2)
    B2 = seg144(Y2, g_dst, s_src, zr144).reshape(2, NP, WX)
    hp3 = post(X2, F2, B2, g2, ln2gr, ln2br, Wout)

    s3 = seg128(hp3, g_src, s_dst, zr128).reshape(2, NP, D)
    emb = pl.pallas_call(
        _tc_final,
        grid=(NBLK,),
        in_specs=[_row(D), _rowp(D), _row(1), _bcast(D)],
        out_specs=_row(D),
        out_shape=_sds((NP, D)),
    )(hp3, s3, dinv, boutr)
    return emb[:N]


# speedup vs baseline: 5.0722x; 1.0099x over previous
import functools

import jax
import jax.numpy as jnp
from jax import lax
from jax.experimental import pallas as pl
from jax.experimental.pallas import tpu as pltpu
from jax.experimental.pallas import tpu_sc as plsc

N = 10000
E = 320000
D = 128
HD = 64
WA = 80
NP = 10240
T = 10.0
ENT_W = 1.0
EPS = 1e-12

NC = 2
NS = 16
NWORK = NC * NS
CHUNK = 128
CH_PER = -(-E // (CHUNK * NWORK) // 8) * 8
CH_TOT = CH_PER * NWORK
EPAD = CH_TOT * CHUNK
GARBAGE = 10200
SLICE = NP // NS
NBUF = 4
DEG_GRP = 8

BR = 256
NBLK = NP // BR


def _sc_segsum(W):
    mesh = plsc.VectorSubcoreMesh(core_axis_name="c", subcore_axis_name="s")

    @functools.partial(
        pl.kernel,
        out_type=jax.ShapeDtypeStruct((2 * NP, W), jnp.float32),
        mesh=mesh,
        compiler_params=pltpu.CompilerParams(use_tc_tiling_on_sc=False),
        scratch_types=[
            pltpu.VMEM((CH_PER, CHUNK), jnp.int32),
            pltpu.VMEM((CH_PER, CHUNK), jnp.int32),
            pltpu.VMEM((CHUNK, W), jnp.float32),
            pltpu.VMEM((CHUNK, W), jnp.float32),
            pltpu.VMEM((CHUNK, W), jnp.float32),
            pltpu.VMEM((CHUNK, W), jnp.float32),
            pltpu.VMEM_SHARED((NP, W), jnp.float32),
            pltpu.SemaphoreType.DMA,
            pltpu.SemaphoreType.DMA,
            pltpu.SemaphoreType.DMA,
            pltpu.SemaphoreType.DMA,
        ],
    )
    def k(table, gidx, sidx, zrows, out, gidx_v, sidx_v, r0, r1, r2, r3,
          acc, g0, g1, g2, g3):
        rows = (r0, r1, r2, r3)
        sems = (g0, g1, g2, g3)
        c = lax.axis_index("c")
        s = lax.axis_index("s")
        tile = c * NS + s
        base = tile * CH_PER
        pltpu.sync_copy(gidx.at[pl.ds(base, CH_PER)], gidx_v)
        pltpu.sync_copy(sidx.at[pl.ds(base, CH_PER)], sidx_v)
        pltpu.sync_copy(zrows, acc.at[pl.ds(s * SLICE, SLICE)])
        plsc.subcore_barrier()

        def grp(g, carry):
            for b in range(NBUF):
                j = g * NBUF + b
                pltpu.async_copy(table.at[gidx_v.at[j]], rows[b], sems[b])
            for b in range(NBUF):
                j = g * NBUF + b
                pltpu.make_async_copy(
                    zrows.at[pl.ds(0, CHUNK)], rows[b], sems[b]).wait()
                pltpu.sync_copy(rows[b], acc.at[sidx_v.at[j]], add=True)
            return carry

        lax.fori_loop(0, CH_PER // NBUF, grp, 0)
        plsc.subcore_barrier()
        pltpu.sync_copy(acc.at[pl.ds(s * SLICE, SLICE)],
                        out.at[pl.ds(c * NP + s * SLICE, SLICE)])

    return k


def _sc_degree():
    mesh = plsc.VectorSubcoreMesh(core_axis_name="c", subcore_axis_name="s")

    @functools.partial(
        pl.kernel,
        out_type=jax.ShapeDtypeStruct((2 * NP, HD), jnp.float32),
        mesh=mesh,
        compiler_params=pltpu.CompilerParams(use_tc_tiling_on_sc=False),
        scratch_types=[
            pltpu.VMEM((CH_PER, CHUNK), jnp.int32),
            pltpu.VMEM((CHUNK, HD), jnp.float32),
            pltpu.VMEM_SHARED((NP, HD), jnp.float32),
            pltpu.SemaphoreType.DMA,
        ],
    )
    def k(ones_hbm, sidx, zrows, out, sidx_v, rows_v, acc, sem):
        c = lax.axis_index("c")
        s = lax.axis_index("s")
        tile = c * NS + s
        base = tile * CH_PER
        pltpu.sync_copy(sidx.at[pl.ds(base, CH_PER)], sidx_v)
        pltpu.sync_copy(ones_hbm, rows_v)
        pltpu.sync_copy(zrows, acc.at[pl.ds(s * SLICE, SLICE)])
        plsc.subcore_barrier()

        def step(g, carry):
            for b in range(DEG_GRP):
                j = g * DEG_GRP + b
                pltpu.async_copy(rows_v, acc.at[sidx_v.at[j]], sem,
                                 add=True)
            for b in range(DEG_GRP):
                pltpu.make_async_copy(ones_hbm, rows_v, sem).wait()
            return carry

        lax.fori_loop(0, CH_PER // DEG_GRP, step, 0)
        plsc.subcore_barrier()
        pltpu.sync_copy(acc.at[pl.ds(s * SLICE, SLICE)],
                        out.at[pl.ds(c * NP + s * SLICE, SLICE)])

    return k


def _tc_prep1(x_ref, w_ref, degp_ref, hpa_ref, hpb_ref, dinv_ref, deg0_ref):
    deg0 = degp_ref[0, :, 0:1] + degp_ref[1, :, 0:1]
    dinv = lax.rsqrt(deg0 + 1.0)
    deg0_ref[...] = deg0
    dinv_ref[...] = dinv
    h = jnp.dot(x_ref[...], w_ref[...], preferred_element_type=jnp.float32)
    h = h * dinv
    hpa_ref[...] = h[:, 0:HD]
    hpb_ref[...] = h[:, HD:D]


def _tc_conv_post(hpa_ref, hpb_ref, spa_ref, spb_ref, dinv_ref, b_ref,
                  xa_ref, xb_ref):
    dinv = dinv_ref[...]
    b = b_ref[...]
    outa = dinv * (spa_ref[0] + spa_ref[1] + hpa_ref[...]) + b[:, 0:HD]
    outb = dinv * (spb_ref[0] + spb_ref[1] + hpb_ref[...]) + b[:, HD:D]
    n = (jnp.sum(outa * outa, axis=1, keepdims=True)
         + jnp.sum(outb * outb, axis=1, keepdims=True))
    xa_ref[...] = jnp.concatenate(
        [outa, n, jnp.zeros((outa.shape[0], WA - HD - 1), jnp.float32)],
        axis=1)
    xb_ref[...] = outb


def _tc_energy(xa_ref, xb_ref, fa_ref, fb_ref, deg0_ref, e_ref):
    xa = xa_ref[...]
    xb = xb_ref[...]
    fa = fa_ref[0] + fa_ref[1]
    fb = fb_ref[0] + fb_ref[1]
    n = xa[:, HD:HD + 1]
    t = fa[:, HD:HD + 1]
    sx = (jnp.sum(fa[:, 0:HD] * xa[:, 0:HD], axis=1, keepdims=True)
          + jnp.sum(fb * xb, axis=1, keepdims=True))
    e_ref[...] = 0.5 * t + 0.5 * deg0_ref[...] * n - sx


def _tc_softgrad(e_ref, g_ref):
    lin = (lax.broadcasted_iota(jnp.int32, (NP // D, D), 0) * D
           + lax.broadcasted_iota(jnp.int32, (NP // D, D), 1))
    mask = lin < N
    e = jnp.where(mask, e_ref[...], 0.0)
    m = jnp.max(e)
    r = 1.0 / (m + EPS)
    z = -(e * r) / T
    zmax = jnp.max(jnp.where(mask, z, -1e30))
    ez = jnp.where(mask, jnp.exp(z - zmax), 0.0)
    p = ez / jnp.sum(ez)
    q = -(jnp.log(p + EPS) + p / (p + EPS))
    pq = jnp.sum(p * q)
    w = -(p * (q - pq)) / T
    onehot = jnp.where(mask & (e == m), 1.0, 0.0)
    cnt = jnp.sum(onehot)
    g_ref[...] = w * r - onehot * (jnp.sum(w * e) * r * r / cnt)


def _tc_ybuild(xa_ref, xb_ref, g_ref, ya_ref, yb_ref):
    ge = g_ref[...]
    ya_ref[...] = jnp.concatenate(
        [ge * xa_ref[:, 0:HD], ge,
         jnp.zeros((ge.shape[0], WA - HD - 1), jnp.float32)], axis=1)
    yb_ref[...] = ge * xb_ref[...]


def _tc_post(xa_ref, xb_ref, fa_ref, fb_ref, ba_ref, bb_ref, g_ref,
             deg0_ref, dinv_ref, lng_ref, lnb_ref, w_ref,
             hpa_ref, hpb_ref):
    xa = xa_ref[...]
    x = jnp.concatenate([xa[:, 0:HD], xb_ref[...]], axis=1)
    fa = fa_ref[0] + fa_ref[1]
    fb = fb_ref[0] + fb_ref[1]
    s = jnp.concatenate([fa[:, 0:HD], fb], axis=1)
    ba = ba_ref[0] + ba_ref[1]
    bb = bb_ref[0] + bb_ref[1]
    cc = jnp.concatenate([ba[:, 0:HD], bb], axis=1)
    a = ba[:, HD:HD + 1]
    ge = g_ref[...]
    grad = x * (a + ge * deg0_ref[...]) - cc - ge * s
    h = x + ENT_W * grad
    h = jnp.maximum(h, 0.0)
    mu = jnp.mean(h, axis=1, keepdims=True)
    d = h - mu
    var = jnp.mean(d * d, axis=1, keepdims=True)
    hn = d / jnp.sqrt(var + 1e-5) * lng_ref[...] + lnb_ref[...]
    h2 = jnp.dot(hn, w_ref[...], preferred_element_type=jnp.float32)
    h2 = h2 * dinv_ref[...]
    hpa_ref[...] = h2[:, 0:HD]
    hpb_ref[...] = h2[:, HD:D]


def _tc_final(hpa_ref, hpb_ref, spa_ref, spb_ref, dinv_ref, b_ref, out_ref):
    dinv = dinv_ref[...]
    b = b_ref[...]
    outa = dinv * (spa_ref[0] + spa_ref[1] + hpa_ref[...]) + b[:, 0:HD]
    outb = dinv * (spb_ref[0] + spb_ref[1] + hpb_ref[...]) + b[:, HD:D]
    out_ref[...] = jnp.concatenate([outa, outb], axis=1)


def _sds(shape):
    return jax.ShapeDtypeStruct(shape, jnp.float32)


def _row(w):
    return pl.BlockSpec((BR, w), lambda i: (i, 0))


def _rowp(w):
    return pl.BlockSpec((2, BR, w), lambda i: (0, i, 0))


def _bcast(w):
    return pl.BlockSpec((1, w), lambda i: (0, 0))


def _mat():
    return pl.BlockSpec((D, D), lambda i: (0, 0))


def kernel(x, edge_index, W1, b1, ln1_g, ln1_b, W2, b2, ln2_g, ln2_b,
           Wout, bout):
    f32 = jnp.float32
    src = edge_index[0].astype(jnp.int32)
    dst = edge_index[1].astype(jnp.int32)
    pe = EPAD - E
    zpad = jnp.zeros((pe,), jnp.int32)
    gpad = jnp.full((pe,), GARBAGE, jnp.int32)
    g_src = jnp.concatenate([src, zpad]).reshape(CH_TOT, CHUNK)
    s_dst = jnp.concatenate([dst, gpad]).reshape(CH_TOT, CHUNK)
    g_dst = jnp.concatenate([dst, zpad]).reshape(CH_TOT, CHUNK)
    s_src = jnp.concatenate([src, gpad]).reshape(CH_TOT, CHUNK)
    zr64 = jnp.zeros((SLICE, HD), f32)
    zr80 = jnp.zeros((SLICE, WA), f32)
    ones64 = jnp.ones((CHUNK, HD), f32)
    xp = jnp.pad(x, ((0, NP - N), (0, 0)))
    b1r = b1.reshape(1, D)
    b2r = b2.reshape(1, D)
    boutr = bout.reshape(1, D)
    ln1gr = ln1_g.reshape(1, D)
    ln1br = ln1_b.reshape(1, D)
    ln2gr = ln2_g.reshape(1, D)
    ln2br = ln2_b.reshape(1, D)

    seg64 = _sc_segsum(HD)
    seg80 = _sc_segsum(WA)

    degp = _sc_degree()(ones64, s_dst, zr64).reshape(2, NP, HD)

    hpa1, hpb1, dinv, deg0 = pl.pallas_call(
        _tc_prep1,
        grid=(NBLK,),
        in_specs=[_row(D), _mat(), _rowp(HD)],
        out_specs=[_row(HD), _row(HD), _row(1), _row(1)],
        out_shape=[_sds((NP, HD)), _sds((NP, HD)), _sds((NP, 1)),
                   _sds((NP, 1))],
    )(xp, W1, degp)

    def conv_seg(hpa, hpb):
        spa = seg64(hpa, g_src, s_dst, zr64).reshape(2, NP, HD)
        spb = seg64(hpb, g_src, s_dst, zr64).reshape(2, NP, HD)
        return spa, spb

    def conv_post(hpa, hpb, spa, spb, br):
        return pl.pallas_call(
            _tc_conv_post,
            grid=(NBLK,),
            in_specs=[_row(HD), _row(HD), _rowp(HD), _rowp(HD), _row(1),
                      _bcast(D)],
            out_specs=[_row(WA), _row(HD)],
            out_shape=[_sds((NP, WA)), _sds((NP, HD))],
        )(hpa, hpb, spa, spb, dinv, br)

    def entropy_grad(xa, xb):
        fa = seg80(xa, g_src, s_dst, zr80).reshape(2, NP, WA)
        fb = seg64(xb, g_src, s_dst, zr64).reshape(2, NP, HD)
        e = pl.pallas_call(
            _tc_energy,
            grid=(NBLK,),
            in_specs=[_row(WA), _row(HD), _rowp(WA), _rowp(HD), _row(1)],
            out_specs=_row(1),
            out_shape=_sds((NP, 1)),
        )(xa, xb, fa, fb, deg0)
        ge = pl.pallas_call(
            _tc_softgrad,
            out_shape=_sds((NP // D, D)),
        )(e.reshape(NP // D, D)).reshape(NP, 1)
        ya, yb = pl.pallas_call(
            _tc_ybuild,
            grid=(NBLK,),
            in_specs=[_row(WA), _row(HD), _row(1)],
            out_specs=[_row(WA), _row(HD)],
            out_shape=[_sds((NP, WA)), _sds((NP, HD))],
        )(xa, xb, ge)
        ba = seg80(ya, g_dst, s_src, zr80).reshape(2, NP, WA)
        bb = seg64(yb, g_dst, s_src, zr64).reshape(2, NP, HD)
        return fa, fb, ba, bb, ge

    def post(xa, xb, fa, fb, ba, bb, ge, lng, lnb, w):
        return pl.pallas_call(
            _tc_post,
            grid=(NBLK,),
            in_specs=[_row(WA), _row(HD), _rowp(WA), _rowp(HD), _rowp(WA),
                      _rowp(HD), _row(1), _row(1), _row(1), _bcast(D),
                      _bcast(D), _mat()],
            out_specs=[_row(HD), _row(HD)],
            out_shape=[_sds((NP, HD)), _sds((NP, HD))],
        )(xa, xb, fa, fb, ba, bb, ge, deg0, dinv, lng, lnb, w)

    spa1, spb1 = conv_seg(hpa1, hpb1)
    xa1, xb1 = conv_post(hpa1, hpb1, spa1, spb1, b1r)
    fa1, fb1, ba1, bb1, g1 = entropy_grad(xa1, xb1)
    hpa2, hpb2 = post(xa1, xb1, fa1, fb1, ba1, bb1, g1, ln1gr, ln1br, W2)

    spa2, spb2 = conv_seg(hpa2, hpb2)
    xa2, xb2 = conv_post(hpa2, hpb2, spa2, spb2, b2r)
    fa2, fb2, ba2, bb2, g2 = entropy_grad(xa2, xb2)
    hpa3, hpb3 = post(xa2, xb2, fa2, fb2, ba2, bb2, g2, ln2gr, ln2br, Wout)

    spa3, spb3 = conv_seg(hpa3, hpb3)
    emb = pl.pallas_call(
        _tc_final,
        grid=(NBLK,),
        in_specs=[_row(HD), _row(HD), _rowp(HD), _rowp(HD), _row(1),
                  _bcast(D)],
        out_specs=_row(D),
        out_shape=_sds((NP, D)),
    )(hpa3, hpb3, spa3, spb3, dinv, boutr)
    return emb[:N]
